# Initial kernel scaffold; baseline (speedup 1.0000x reference)
import jax
import jax.numpy as jnp
from jax.experimental import pallas as pl


def kernel(x, routing_tensor, W1, b1, W2, b2):
    raise NotImplementedError("write your pallas kernel here")


# speedup vs baseline: 4.0814x; 4.0814x over previous
import functools

import jax
import jax.numpy as jnp
from jax import lax
from jax.experimental import pallas as pl
from jax.experimental.pallas import tpu as pltpu
from jax.experimental.pallas import tpu_sc as plsc

DIM = 768
NUM_EXPERTS = 8
EXPERT_DIM = 1536
TOKENS = 4096
TOP_K = 2
PAIRS = TOKENS * TOP_K
MB = 512
NB = PAIRS // MB
MAX_TILES = NB + NUM_EXPERTS - 1

NCORES = 2
NSUB = 16
NW = NCORES * NSUB
TPW = TOKENS // NW
SUB = 64

def _sc_dispatch(x, pos):
    mesh = plsc.VectorSubcoreMesh(core_axis_name="c", subcore_axis_name="s")

    @functools.partial(
        pl.kernel,
        out_type=jax.ShapeDtypeStruct((PAIRS, DIM), jnp.float32),
        mesh=mesh,
        scratch_types=[
            pltpu.VMEM((SUB,), jnp.int32),
            pltpu.VMEM((SUB,), jnp.int32),
            pltpu.VMEM((SUB, DIM), jnp.float32),
            pltpu.SemaphoreType.DMA,
        ],
    )
    def body(x_hbm, pos_hbm, xs_hbm, idx0_v, idx1_v, rows_v, sem):
        wid = lax.axis_index("s") * NCORES + lax.axis_index("c")
        base = wid * TPW
        for j in range(TPW // SUB):
            b = base + j * SUB
            pltpu.sync_copy(pos_hbm.at[pl.ds(b, SUB)], idx0_v)
            pltpu.sync_copy(pos_hbm.at[pl.ds(TOKENS + b, SUB)], idx1_v)
            pltpu.sync_copy(x_hbm.at[pl.ds(b, SUB)], rows_v)
            pltpu.async_copy(rows_v, xs_hbm.at[idx0_v], sem).wait()
            pltpu.async_copy(rows_v, xs_hbm.at[idx1_v], sem).wait()

    return body(x, pos)


def _sc_combine(ys, pos, vals0, vals1):
    mesh = plsc.VectorSubcoreMesh(core_axis_name="c", subcore_axis_name="s")

    @functools.partial(
        pl.kernel,
        out_type=jax.ShapeDtypeStruct((TOKENS, DIM), jnp.float32),
        mesh=mesh,
        scratch_types=[
            pltpu.VMEM((SUB,), jnp.int32),
            pltpu.VMEM((SUB,), jnp.int32),
            pltpu.VMEM((SUB,), jnp.float32),
            pltpu.VMEM((SUB,), jnp.float32),
            pltpu.VMEM((SUB, DIM), jnp.float32),
            pltpu.VMEM((SUB, DIM), jnp.float32),
            pltpu.SemaphoreType.DMA,
        ],
    )
    def body(ys_hbm, pos_hbm, vals0_hbm, vals1_hbm, out_hbm,
             idx0_v, idx1_v, vv0, vv1, r0, r1, sem):
        wid = lax.axis_index("s") * NCORES + lax.axis_index("c")
        base = wid * TPW

        def chunk(j, carry):
            b = base + j * SUB
            pltpu.sync_copy(pos_hbm.at[pl.ds(b, SUB)], idx0_v)
            pltpu.sync_copy(pos_hbm.at[pl.ds(TOKENS + b, SUB)], idx1_v)
            pltpu.sync_copy(vals0_hbm.at[pl.ds(b, SUB)], vv0)
            pltpu.sync_copy(vals1_hbm.at[pl.ds(b, SUB)], vv1)
            pltpu.async_copy(ys_hbm.at[idx0_v], r0, sem).wait()
            pltpu.async_copy(ys_hbm.at[idx1_v], r1, sem).wait()

            def inner(g, c2):
                wv0 = vv0[pl.ds(g * 16, 16)]
                wv1 = vv1[pl.ds(g * 16, 16)]
                for k in range(16):
                    w0 = wv0[k]
                    w1 = wv1[k]
                    t = g * 16 + k
                    for v in range(DIM // 16):
                        sl = pl.ds(v * 16, 16)
                        r0[t, sl] = r0[t, sl] * w0 + r1[t, sl] * w1
                return c2

            lax.fori_loop(0, SUB // 16, inner, 0)
            pltpu.sync_copy(r0, out_hbm.at[pl.ds(b, SUB)])
            return carry

        lax.fori_loop(0, TPW // SUB, chunk, 0)

    return body(ys, pos, vals0, vals1)


def _gmm_kernel(tb_ref, te_ref, tf_ref, tlo_ref, thi_ref,
                xs_ref, w1_ref, b1_ref, w2_ref, b2_ref, ys_ref):
    t = pl.program_id(0)
    row0 = tb_ref[t] * MB
    g = row0 + lax.broadcasted_iota(jnp.int32, (MB, 1), 0)
    inseg = (g >= tlo_ref[t]) & (g < thi_ref[t])

    xb = xs_ref[...].astype(jnp.bfloat16)
    w1 = w1_ref[0].astype(jnp.bfloat16)
    w2 = w2_ref[0].astype(jnp.bfloat16)
    h = lax.dot_general(xb, w1, (((1,), (0,)), ((), ())),
                        preferred_element_type=jnp.float32) + b1_ref[0, 0][None, :]
    h = (h * 0.5 * (1.0 + lax.erf(h * 0.7071067811865476))).astype(jnp.bfloat16)
    y = lax.dot_general(h, w2, (((1,), (0,)), ((), ())),
                        preferred_element_type=jnp.float32) + b2_ref[0, 0][None, :]
    contrib = jnp.where(inseg, y, 0.0)

    @pl.when(tf_ref[t] == 1)
    def _init():
        ys_ref[...] = contrib

    @pl.when(tf_ref[t] == 0)
    def _acc():
        ys_ref[...] = ys_ref[...] + contrib


def _grouped_mlp(tb, te, tf, tlo, thi, xs, W1, b1, W2, b2):
    grid_spec = pltpu.PrefetchScalarGridSpec(
        num_scalar_prefetch=5,
        grid=(MAX_TILES,),
        in_specs=[
            pl.BlockSpec((MB, DIM), lambda t, tb, te, tf, tlo, thi: (tb[t], 0)),
            pl.BlockSpec((1, DIM, EXPERT_DIM),
                         lambda t, tb, te, tf, tlo, thi: (te[t], 0, 0)),
            pl.BlockSpec((1, 1, EXPERT_DIM),
                         lambda t, tb, te, tf, tlo, thi: (te[t], 0, 0)),
            pl.BlockSpec((1, EXPERT_DIM, DIM),
                         lambda t, tb, te, tf, tlo, thi: (te[t], 0, 0)),
            pl.BlockSpec((1, 1, DIM),
                         lambda t, tb, te, tf, tlo, thi: (te[t], 0, 0)),
        ],
        out_specs=pl.BlockSpec((MB, DIM), lambda t, tb, te, tf, tlo, thi: (tb[t], 0)),
    )
    return pl.pallas_call(
        _gmm_kernel,
        grid_spec=grid_spec,
        out_shape=jax.ShapeDtypeStruct((PAIRS, DIM), jnp.float32),
    )(tb, te, tf, tlo, thi, xs, W1, b1[:, None, :], W2, b2[:, None, :])


def kernel(x, routing_tensor, W1, b1, W2, b2):
    vals, eidx = lax.top_k(routing_tensor, TOP_K)
    e_flat = eidx.astype(jnp.int32).T.reshape(PAIRS)
    onehot = (e_flat[:, None] == jnp.arange(NUM_EXPERTS, dtype=jnp.int32)[None, :]
              ).astype(jnp.int32)
    counts = jnp.sum(onehot, axis=0)
    offsets = jnp.concatenate(
        [jnp.zeros((1,), jnp.int32), jnp.cumsum(counts)[:-1].astype(jnp.int32)])
    ends = offsets + counts
    csum = jnp.cumsum(onehot, axis=0)
    rank = jnp.sum(onehot * (csum - 1), axis=1)
    pos = (rank + jnp.sum(onehot * offsets[None, :], axis=1)).astype(jnp.int32)

    b_arr = jnp.arange(NB, dtype=jnp.int32)
    ov = ((offsets[None, :] < (b_arr[:, None] + 1) * MB)
          & (ends[None, :] > b_arr[:, None] * MB)
          & (counts[None, :] > 0))
    ovf = ov.reshape(-1)
    tidx = jnp.cumsum(ovf.astype(jnp.int32)) - 1
    ntiles = tidx[-1] + 1
    flat = jnp.arange(NB * NUM_EXPERTS, dtype=jnp.int32)
    scat = jnp.where(ovf, tidx, MAX_TILES)
    tb = jnp.zeros((MAX_TILES,), jnp.int32).at[scat].set(
        flat // NUM_EXPERTS, mode="drop")
    te = jnp.zeros((MAX_TILES,), jnp.int32).at[scat].set(
        flat % NUM_EXPERTS, mode="drop")
    tlo = jnp.zeros((MAX_TILES,), jnp.int32).at[scat].set(
        jnp.broadcast_to(offsets[None, :], (NB, NUM_EXPERTS)).reshape(-1),
        mode="drop")
    thi = jnp.zeros((MAX_TILES,), jnp.int32).at[scat].set(
        jnp.broadcast_to(ends[None, :], (NB, NUM_EXPERTS)).reshape(-1),
        mode="drop")
    slot = jnp.arange(MAX_TILES, dtype=jnp.int32)
    pad = slot >= ntiles
    last_b = jnp.take(tb, ntiles - 1)
    last_e = jnp.take(te, ntiles - 1)
    tb = jnp.where(pad, last_b, tb)
    te = jnp.where(pad, last_e, te)
    tlo = jnp.where(pad, 0, tlo)
    thi = jnp.where(pad, 0, thi)
    tf = jnp.concatenate(
        [jnp.ones((1,), jnp.int32),
         (tb[1:] != tb[:-1]).astype(jnp.int32)])

    xs = _sc_dispatch(x, pos)
    ys = _grouped_mlp(tb, te, tf, tlo, thi, xs, W1, b1, W2, b2)
    return _sc_combine(ys, pos, vals[:, 0], vals[:, 1])


# speedup vs baseline: 4.4290x; 1.0852x over previous
import functools

import jax
import jax.numpy as jnp
from jax import lax
from jax.experimental import pallas as pl
from jax.experimental.pallas import tpu as pltpu
from jax.experimental.pallas import tpu_sc as plsc

DIM = 768
NUM_EXPERTS = 8
EXPERT_DIM = 1536
TOKENS = 4096
TOP_K = 2
PAIRS = TOKENS * TOP_K
MB = 256
NB = PAIRS // MB
MAX_TILES = NB + NUM_EXPERTS - 1

NCORES = 2
NSUB = 16
NW = NCORES * NSUB
TPW = TOKENS // NW
SUB = 64

def _sc_dispatch(x, pos, w0, w1):
    mesh = plsc.VectorSubcoreMesh(core_axis_name="c", subcore_axis_name="s")

    @functools.partial(
        pl.kernel,
        out_type=(jax.ShapeDtypeStruct((PAIRS, DIM), jnp.float32),
                  jax.ShapeDtypeStruct((PAIRS, 128), jnp.float32)),
        mesh=mesh,
        scratch_types=[
            pltpu.VMEM((SUB,), jnp.int32),
            pltpu.VMEM((SUB,), jnp.int32),
            pltpu.VMEM((SUB, DIM), jnp.float32),
            pltpu.VMEM((SUB,), jnp.float32),
            pltpu.VMEM((SUB,), jnp.float32),
            pltpu.VMEM((SUB, 128), jnp.float32),
            pltpu.VMEM((SUB, 128), jnp.float32),
            pltpu.SemaphoreType.DMA,
        ],
    )
    def body(x_hbm, pos_hbm, w0_hbm, w1_hbm, xs_hbm, ws_hbm,
             idx0_v, idx1_v, rows_v, wv0, wv1, wrow0, wrow1, sem):
        wid = lax.axis_index("s") * NCORES + lax.axis_index("c")
        base = wid * TPW
        for j in range(TPW // SUB):
            b = base + j * SUB
            pltpu.sync_copy(pos_hbm.at[pl.ds(b, SUB)], idx0_v)
            pltpu.sync_copy(pos_hbm.at[pl.ds(TOKENS + b, SUB)], idx1_v)
            pltpu.sync_copy(x_hbm.at[pl.ds(b, SUB)], rows_v)
            pltpu.sync_copy(w0_hbm.at[pl.ds(b, SUB)], wv0)
            pltpu.sync_copy(w1_hbm.at[pl.ds(b, SUB)], wv1)

            def fill(g, carry):
                g16 = g * 16
                a0 = wv0[pl.ds(g16, 16)]
                a1 = wv1[pl.ds(g16, 16)]
                for k in range(16):
                    for v in range(8):
                        sl = pl.ds(v * 16, 16)
                        wrow0[g16 + k, sl] = jnp.broadcast_to(a0[k], (16,))
                        wrow1[g16 + k, sl] = jnp.broadcast_to(a1[k], (16,))
                return carry

            lax.fori_loop(0, SUB // 16, fill, 0)
            c0 = pltpu.async_copy(rows_v, xs_hbm.at[idx0_v], sem)
            c1 = pltpu.async_copy(rows_v, xs_hbm.at[idx1_v], sem)
            c2 = pltpu.async_copy(wrow0, ws_hbm.at[idx0_v], sem)
            c3 = pltpu.async_copy(wrow1, ws_hbm.at[idx1_v], sem)
            c0.wait()
            c1.wait()
            c2.wait()
            c3.wait()

    return body(x, pos, w0, w1)


def _sc_combine(ys, pos):
    mesh = plsc.VectorSubcoreMesh(core_axis_name="c", subcore_axis_name="s")

    @functools.partial(
        pl.kernel,
        out_type=jax.ShapeDtypeStruct((TOKENS, DIM), jnp.float32),
        mesh=mesh,
        scratch_types=[
            pltpu.VMEM((SUB,), jnp.int32),
            pltpu.VMEM((SUB,), jnp.int32),
            pltpu.VMEM((SUB, DIM), jnp.float32),
            pltpu.VMEM((SUB, DIM), jnp.float32),
            pltpu.SemaphoreType.DMA,
        ],
    )
    def body(ys_hbm, pos_hbm, out_hbm, idx0_v, idx1_v, r0, r1, sem):
        wid = lax.axis_index("s") * NCORES + lax.axis_index("c")
        base = wid * TPW

        def chunk(j, carry):
            b = base + j * SUB
            pltpu.sync_copy(pos_hbm.at[pl.ds(b, SUB)], idx0_v)
            pltpu.sync_copy(pos_hbm.at[pl.ds(TOKENS + b, SUB)], idx1_v)
            c0 = pltpu.async_copy(ys_hbm.at[idx0_v], r0, sem)
            c1 = pltpu.async_copy(ys_hbm.at[idx1_v], r1, sem)
            c0.wait()
            c1.wait()

            def inner(t, c2):
                for v in range(DIM // 16):
                    sl = pl.ds(v * 16, 16)
                    plsc.addupdate(r0.at[t, sl], r1[t, sl])
                return c2

            lax.fori_loop(0, SUB, inner, 0)
            pltpu.sync_copy(r0, out_hbm.at[pl.ds(b, SUB)])
            return carry

        lax.fori_loop(0, TPW // SUB, chunk, 0)

    return body(ys, pos)


def _gmm_kernel(tb_ref, te_ref, tf_ref, tlo_ref, thi_ref,
                xs_ref, w1_ref, b1_ref, w2_ref, b2_ref, ws_ref, ys_ref):
    t = pl.program_id(0)
    row0 = tb_ref[t] * MB
    g = row0 + lax.broadcasted_iota(jnp.int32, (MB, 1), 0)
    inseg = (g >= tlo_ref[t]) & (g < thi_ref[t])

    xb = xs_ref[...].astype(jnp.bfloat16)
    w1 = w1_ref[0].astype(jnp.bfloat16)
    w2 = w2_ref[0].astype(jnp.bfloat16)
    h = lax.dot_general(xb, w1, (((1,), (0,)), ((), ())),
                        preferred_element_type=jnp.float32) + b1_ref[0, 0][None, :]
    h = (h * 0.5 * (1.0 + lax.erf(h * 0.7071067811865476))).astype(jnp.bfloat16)
    y = lax.dot_general(h, w2, (((1,), (0,)), ((), ())),
                        preferred_element_type=jnp.float32) + b2_ref[0, 0][None, :]
    contrib = jnp.where(inseg, y * ws_ref[:, 0:1], 0.0)

    @pl.when(tf_ref[t] == 1)
    def _init():
        ys_ref[...] = contrib

    @pl.when(tf_ref[t] == 0)
    def _acc():
        ys_ref[...] = ys_ref[...] + contrib


def _grouped_mlp(tb, te, tf, tlo, thi, xs, W1, b1, W2, b2, ws):
    grid_spec = pltpu.PrefetchScalarGridSpec(
        num_scalar_prefetch=5,
        grid=(MAX_TILES,),
        in_specs=[
            pl.BlockSpec((MB, DIM), lambda t, tb, te, tf, tlo, thi: (tb[t], 0)),
            pl.BlockSpec((1, DIM, EXPERT_DIM),
                         lambda t, tb, te, tf, tlo, thi: (te[t], 0, 0)),
            pl.BlockSpec((1, 1, EXPERT_DIM),
                         lambda t, tb, te, tf, tlo, thi: (te[t], 0, 0)),
            pl.BlockSpec((1, EXPERT_DIM, DIM),
                         lambda t, tb, te, tf, tlo, thi: (te[t], 0, 0)),
            pl.BlockSpec((1, 1, DIM),
                         lambda t, tb, te, tf, tlo, thi: (te[t], 0, 0)),
            pl.BlockSpec((MB, 128), lambda t, tb, te, tf, tlo, thi: (tb[t], 0)),
        ],
        out_specs=pl.BlockSpec((MB, DIM), lambda t, tb, te, tf, tlo, thi: (tb[t], 0)),
    )
    return pl.pallas_call(
        _gmm_kernel,
        grid_spec=grid_spec,
        out_shape=jax.ShapeDtypeStruct((PAIRS, DIM), jnp.float32),
    )(tb, te, tf, tlo, thi, xs, W1, b1[:, None, :], W2, b2[:, None, :], ws)


def kernel(x, routing_tensor, W1, b1, W2, b2):
    vals, eidx = lax.top_k(routing_tensor, TOP_K)
    e_flat = eidx.astype(jnp.int32).T.reshape(PAIRS)
    onehot = (e_flat[:, None] == jnp.arange(NUM_EXPERTS, dtype=jnp.int32)[None, :]
              ).astype(jnp.int32)
    counts = jnp.sum(onehot, axis=0)
    offsets = jnp.concatenate(
        [jnp.zeros((1,), jnp.int32), jnp.cumsum(counts)[:-1].astype(jnp.int32)])
    ends = offsets + counts
    csum = jnp.cumsum(onehot, axis=0)
    rank = jnp.sum(onehot * (csum - 1), axis=1)
    pos = (rank + jnp.sum(onehot * offsets[None, :], axis=1)).astype(jnp.int32)

    b_arr = jnp.arange(NB, dtype=jnp.int32)
    ov = ((offsets[None, :] < (b_arr[:, None] + 1) * MB)
          & (ends[None, :] > b_arr[:, None] * MB)
          & (counts[None, :] > 0))
    ovf = ov.reshape(-1)
    tidx = jnp.cumsum(ovf.astype(jnp.int32)) - 1
    ntiles = tidx[-1] + 1
    flat = jnp.arange(NB * NUM_EXPERTS, dtype=jnp.int32)
    scat = jnp.where(ovf, tidx, MAX_TILES)
    tb = jnp.zeros((MAX_TILES,), jnp.int32).at[scat].set(
        flat // NUM_EXPERTS, mode="drop")
    te = jnp.zeros((MAX_TILES,), jnp.int32).at[scat].set(
        flat % NUM_EXPERTS, mode="drop")
    tlo = jnp.zeros((MAX_TILES,), jnp.int32).at[scat].set(
        jnp.broadcast_to(offsets[None, :], (NB, NUM_EXPERTS)).reshape(-1),
        mode="drop")
    thi = jnp.zeros((MAX_TILES,), jnp.int32).at[scat].set(
        jnp.broadcast_to(ends[None, :], (NB, NUM_EXPERTS)).reshape(-1),
        mode="drop")
    slot = jnp.arange(MAX_TILES, dtype=jnp.int32)
    pad = slot >= ntiles
    last_b = jnp.take(tb, ntiles - 1)
    last_e = jnp.take(te, ntiles - 1)
    tb = jnp.where(pad, last_b, tb)
    te = jnp.where(pad, last_e, te)
    tlo = jnp.where(pad, 0, tlo)
    thi = jnp.where(pad, 0, thi)
    tf = jnp.concatenate(
        [jnp.ones((1,), jnp.int32),
         (tb[1:] != tb[:-1]).astype(jnp.int32)])

    xs, ws_plane = _sc_dispatch(x, pos, vals[:, 0], vals[:, 1])
    ys = _grouped_mlp(tb, te, tf, tlo, thi, xs, W1, b1, W2, b2, ws_plane)
    return _sc_combine(ys, pos)


# speedup vs baseline: 4.6195x; 1.0430x over previous
import functools

import jax
import jax.numpy as jnp
from jax import lax
from jax.experimental import pallas as pl
from jax.experimental.pallas import tpu as pltpu
from jax.experimental.pallas import tpu_sc as plsc

DIM = 768
NUM_EXPERTS = 8
EXPERT_DIM = 1536
TOKENS = 4096
TOP_K = 2
PAIRS = TOKENS * TOP_K
MB = 512
NB = PAIRS // MB
MAX_TILES = NB + NUM_EXPERTS - 1

NCORES = 2
NSUB = 16
NW = NCORES * NSUB
TPW = TOKENS // NW
SUB = 64

def _sc_dispatch(x, pos, w0, w1):
    mesh = plsc.VectorSubcoreMesh(core_axis_name="c", subcore_axis_name="s")

    @functools.partial(
        pl.kernel,
        out_type=(jax.ShapeDtypeStruct((PAIRS, DIM), jnp.float32),
                  jax.ShapeDtypeStruct((PAIRS, 128), jnp.float32)),
        mesh=mesh,
        scratch_types=[
            pltpu.VMEM((SUB,), jnp.int32),
            pltpu.VMEM((SUB,), jnp.int32),
            pltpu.VMEM((SUB, DIM), jnp.float32),
            pltpu.VMEM((SUB,), jnp.float32),
            pltpu.VMEM((SUB,), jnp.float32),
            pltpu.VMEM((SUB, 128), jnp.float32),
            pltpu.VMEM((SUB, 128), jnp.float32),
            pltpu.SemaphoreType.DMA,
        ],
    )
    def body(x_hbm, pos_hbm, w0_hbm, w1_hbm, xs_hbm, ws_hbm,
             idx0_v, idx1_v, rows_v, wv0, wv1, wrow0, wrow1, sem):
        wid = lax.axis_index("s") * NCORES + lax.axis_index("c")
        base = wid * TPW
        for j in range(TPW // SUB):
            b = base + j * SUB
            pltpu.sync_copy(pos_hbm.at[pl.ds(b, SUB)], idx0_v)
            pltpu.sync_copy(pos_hbm.at[pl.ds(TOKENS + b, SUB)], idx1_v)
            pltpu.sync_copy(x_hbm.at[pl.ds(b, SUB)], rows_v)
            pltpu.sync_copy(w0_hbm.at[pl.ds(b, SUB)], wv0)
            pltpu.sync_copy(w1_hbm.at[pl.ds(b, SUB)], wv1)

            def fill(g, carry):
                g16 = g * 16
                a0 = wv0[pl.ds(g16, 16)]
                a1 = wv1[pl.ds(g16, 16)]
                for k in range(16):
                    for v in range(8):
                        sl = pl.ds(v * 16, 16)
                        wrow0[g16 + k, sl] = jnp.broadcast_to(a0[k], (16,))
                        wrow1[g16 + k, sl] = jnp.broadcast_to(a1[k], (16,))
                return carry

            lax.fori_loop(0, SUB // 16, fill, 0)
            c0 = pltpu.async_copy(rows_v, xs_hbm.at[idx0_v], sem)
            c1 = pltpu.async_copy(rows_v, xs_hbm.at[idx1_v], sem)
            c2 = pltpu.async_copy(wrow0, ws_hbm.at[idx0_v], sem)
            c3 = pltpu.async_copy(wrow1, ws_hbm.at[idx1_v], sem)
            c0.wait()
            c1.wait()
            c2.wait()
            c3.wait()

    return body(x, pos, w0, w1)


def _sc_combine(ys, pos):
    mesh = plsc.VectorSubcoreMesh(core_axis_name="c", subcore_axis_name="s")

    @functools.partial(
        pl.kernel,
        out_type=jax.ShapeDtypeStruct((TOKENS, DIM), jnp.float32),
        mesh=mesh,
        scratch_types=[
            pltpu.VMEM((SUB,), jnp.int32),
            pltpu.VMEM((SUB,), jnp.int32),
            pltpu.VMEM((SUB, DIM), jnp.float32),
            pltpu.VMEM((SUB, DIM), jnp.float32),
            pltpu.SemaphoreType.DMA,
        ],
    )
    def body(ys_hbm, pos_hbm, out_hbm, idx0_v, idx1_v, r0, r1, sem):
        wid = lax.axis_index("s") * NCORES + lax.axis_index("c")
        base = wid * TPW

        def chunk(j, carry):
            b = base + j * SUB
            pltpu.sync_copy(pos_hbm.at[pl.ds(b, SUB)], idx0_v)
            pltpu.sync_copy(pos_hbm.at[pl.ds(TOKENS + b, SUB)], idx1_v)
            c0 = pltpu.async_copy(ys_hbm.at[idx0_v], r0, sem)
            c1 = pltpu.async_copy(ys_hbm.at[idx1_v], r1, sem)
            c0.wait()
            c1.wait()

            def inner(t, c2):
                for v in range(DIM // 16):
                    sl = pl.ds(v * 16, 16)
                    plsc.addupdate(r0.at[t, sl], r1[t, sl])
                return c2

            lax.fori_loop(0, SUB, inner, 0)
            pltpu.sync_copy(r0, out_hbm.at[pl.ds(b, SUB)])
            return carry

        lax.fori_loop(0, TPW // SUB, chunk, 0)

    return body(ys, pos)


def _gmm_kernel(tb_ref, te_ref, tf_ref, tlo_ref, thi_ref,
                xs_ref, w1_ref, b1_ref, w2_ref, b2_ref, ws_ref, ys_ref):
    t = pl.program_id(0)
    row0 = tb_ref[t] * MB
    g = row0 + lax.broadcasted_iota(jnp.int32, (MB, 1), 0)
    inseg = (g >= tlo_ref[t]) & (g < thi_ref[t])

    xb = xs_ref[...].astype(jnp.bfloat16)
    w1 = w1_ref[0].astype(jnp.bfloat16)
    w2 = w2_ref[0].astype(jnp.bfloat16)
    h = lax.dot_general(xb, w1, (((1,), (0,)), ((), ())),
                        preferred_element_type=jnp.float32) + b1_ref[0, 0][None, :]
    h = (h * 0.5 * (1.0 + lax.erf(h * 0.7071067811865476))).astype(jnp.bfloat16)
    y = lax.dot_general(h, w2, (((1,), (0,)), ((), ())),
                        preferred_element_type=jnp.float32) + b2_ref[0, 0][None, :]
    contrib = jnp.where(inseg, y * ws_ref[:, 0:1], 0.0)

    @pl.when(tf_ref[t] == 1)
    def _init():
        ys_ref[...] = contrib

    @pl.when(tf_ref[t] == 0)
    def _acc():
        ys_ref[...] = ys_ref[...] + contrib


def _grouped_mlp(tb, te, tf, tlo, thi, xs, W1, b1, W2, b2, ws):
    grid_spec = pltpu.PrefetchScalarGridSpec(
        num_scalar_prefetch=5,
        grid=(MAX_TILES,),
        in_specs=[
            pl.BlockSpec((MB, DIM), lambda t, tb, te, tf, tlo, thi: (tb[t], 0)),
            pl.BlockSpec((1, DIM, EXPERT_DIM),
                         lambda t, tb, te, tf, tlo, thi: (te[t], 0, 0)),
            pl.BlockSpec((1, 1, EXPERT_DIM),
                         lambda t, tb, te, tf, tlo, thi: (te[t], 0, 0)),
            pl.BlockSpec((1, EXPERT_DIM, DIM),
                         lambda t, tb, te, tf, tlo, thi: (te[t], 0, 0)),
            pl.BlockSpec((1, 1, DIM),
                         lambda t, tb, te, tf, tlo, thi: (te[t], 0, 0)),
            pl.BlockSpec((MB, 128), lambda t, tb, te, tf, tlo, thi: (tb[t], 0)),
        ],
        out_specs=pl.BlockSpec((MB, DIM), lambda t, tb, te, tf, tlo, thi: (tb[t], 0)),
    )
    return pl.pallas_call(
        _gmm_kernel,
        grid_spec=grid_spec,
        out_shape=jax.ShapeDtypeStruct((PAIRS, DIM), jnp.float32),
    )(tb, te, tf, tlo, thi, xs, W1, b1[:, None, :], W2, b2[:, None, :], ws)


def kernel(x, routing_tensor, W1, b1, W2, b2):
    lane = jnp.arange(NUM_EXPERTS, dtype=jnp.int32)[None, :]
    e0 = jnp.argmax(routing_tensor, axis=1).astype(jnp.int32)
    v0 = jnp.max(routing_tensor, axis=1)
    rt2 = jnp.where(lane == e0[:, None], -jnp.inf, routing_tensor)
    e1 = jnp.argmax(rt2, axis=1).astype(jnp.int32)
    v1 = jnp.max(rt2, axis=1)
    vals = jnp.stack([v0, v1], axis=1)
    eidx = jnp.stack([e0, e1], axis=1)
    e_flat = eidx.astype(jnp.int32).T.reshape(PAIRS)
    onehot = (e_flat[:, None] == jnp.arange(NUM_EXPERTS, dtype=jnp.int32)[None, :]
              ).astype(jnp.int32)
    counts = jnp.sum(onehot, axis=0)
    offsets = jnp.concatenate(
        [jnp.zeros((1,), jnp.int32), jnp.cumsum(counts)[:-1].astype(jnp.int32)])
    ends = offsets + counts
    csum = jnp.cumsum(onehot, axis=0)
    rank = jnp.sum(onehot * (csum - 1), axis=1)
    pos = (rank + jnp.sum(onehot * offsets[None, :], axis=1)).astype(jnp.int32)

    b_arr = jnp.arange(NB, dtype=jnp.int32)
    ov = ((offsets[None, :] < (b_arr[:, None] + 1) * MB)
          & (ends[None, :] > b_arr[:, None] * MB)
          & (counts[None, :] > 0))
    ovf = ov.reshape(-1)
    tidx = jnp.cumsum(ovf.astype(jnp.int32)) - 1
    ntiles = tidx[-1] + 1
    flat = jnp.arange(NB * NUM_EXPERTS, dtype=jnp.int32)
    scat = jnp.where(ovf, tidx, MAX_TILES)
    tb = jnp.zeros((MAX_TILES,), jnp.int32).at[scat].set(
        flat // NUM_EXPERTS, mode="drop")
    te = jnp.zeros((MAX_TILES,), jnp.int32).at[scat].set(
        flat % NUM_EXPERTS, mode="drop")
    tlo = jnp.zeros((MAX_TILES,), jnp.int32).at[scat].set(
        jnp.broadcast_to(offsets[None, :], (NB, NUM_EXPERTS)).reshape(-1),
        mode="drop")
    thi = jnp.zeros((MAX_TILES,), jnp.int32).at[scat].set(
        jnp.broadcast_to(ends[None, :], (NB, NUM_EXPERTS)).reshape(-1),
        mode="drop")
    slot = jnp.arange(MAX_TILES, dtype=jnp.int32)
    pad = slot >= ntiles
    last_b = jnp.take(tb, ntiles - 1)
    last_e = jnp.take(te, ntiles - 1)
    tb = jnp.where(pad, last_b, tb)
    te = jnp.where(pad, last_e, te)
    tlo = jnp.where(pad, 0, tlo)
    thi = jnp.where(pad, 0, thi)
    tf = jnp.concatenate(
        [jnp.ones((1,), jnp.int32),
         (tb[1:] != tb[:-1]).astype(jnp.int32)])

    xs, ws_plane = _sc_dispatch(x, pos, vals[:, 0], vals[:, 1])
    ys = _grouped_mlp(tb, te, tf, tlo, thi, xs, W1, b1, W2, b2, ws_plane)
    return _sc_combine(ys, pos)


# speedup vs baseline: 4.9019x; 1.0611x over previous
import functools

import jax
import jax.numpy as jnp
from jax import lax
from jax.experimental import pallas as pl
from jax.experimental.pallas import tpu as pltpu
from jax.experimental.pallas import tpu_sc as plsc

DIM = 768
NUM_EXPERTS = 8
EXPERT_DIM = 1536
TOKENS = 4096
TOP_K = 2
PAIRS = TOKENS * TOP_K
MB = 512
NB = PAIRS // MB
NBP = NB + NUM_EXPERTS
PADROWS = NBP * MB

NCORES = 2
NSUB = 16
NW = NCORES * NSUB
TPW = TOKENS // NW
SUB = 64

def _sc_dispatch(x, pos, w0, w1):
    mesh = plsc.VectorSubcoreMesh(core_axis_name="c", subcore_axis_name="s")

    @functools.partial(
        pl.kernel,
        out_type=(jax.ShapeDtypeStruct((PADROWS, DIM), jnp.float32),
                  jax.ShapeDtypeStruct((PADROWS, 128), jnp.float32)),
        mesh=mesh,
        scratch_types=[
            pltpu.VMEM((SUB,), jnp.int32),
            pltpu.VMEM((SUB,), jnp.int32),
            pltpu.VMEM((SUB, DIM), jnp.float32),
            pltpu.VMEM((SUB,), jnp.float32),
            pltpu.VMEM((SUB,), jnp.float32),
            pltpu.VMEM((SUB, 128), jnp.float32),
            pltpu.VMEM((SUB, 128), jnp.float32),
            pltpu.SemaphoreType.DMA,
        ],
    )
    def body(x_hbm, pos_hbm, w0_hbm, w1_hbm, xs_hbm, ws_hbm,
             idx0_v, idx1_v, rows_v, wv0, wv1, wrow0, wrow1, sem):
        wid = lax.axis_index("s") * NCORES + lax.axis_index("c")
        base = wid * TPW
        for j in range(TPW // SUB):
            b = base + j * SUB
            pltpu.sync_copy(pos_hbm.at[pl.ds(b, SUB)], idx0_v)
            pltpu.sync_copy(pos_hbm.at[pl.ds(TOKENS + b, SUB)], idx1_v)
            pltpu.sync_copy(x_hbm.at[pl.ds(b, SUB)], rows_v)
            pltpu.sync_copy(w0_hbm.at[pl.ds(b, SUB)], wv0)
            pltpu.sync_copy(w1_hbm.at[pl.ds(b, SUB)], wv1)

            def fill(g, carry):
                g16 = g * 16
                a0 = wv0[pl.ds(g16, 16)]
                a1 = wv1[pl.ds(g16, 16)]
                for k in range(16):
                    for v in range(8):
                        sl = pl.ds(v * 16, 16)
                        wrow0[g16 + k, sl] = jnp.broadcast_to(a0[k], (16,))
                        wrow1[g16 + k, sl] = jnp.broadcast_to(a1[k], (16,))
                return carry

            lax.fori_loop(0, SUB // 16, fill, 0)
            c0 = pltpu.async_copy(rows_v, xs_hbm.at[idx0_v], sem)
            c1 = pltpu.async_copy(rows_v, xs_hbm.at[idx1_v], sem)
            c2 = pltpu.async_copy(wrow0, ws_hbm.at[idx0_v], sem)
            c3 = pltpu.async_copy(wrow1, ws_hbm.at[idx1_v], sem)
            c0.wait()
            c1.wait()
            c2.wait()
            c3.wait()

    return body(x, pos, w0, w1)


CSUB = 32


def _sc_combine(ys, pos):
    mesh = plsc.VectorSubcoreMesh(core_axis_name="c", subcore_axis_name="s")
    nchunks = TPW // CSUB

    @functools.partial(
        pl.kernel,
        out_type=jax.ShapeDtypeStruct((TOKENS, DIM), jnp.float32),
        mesh=mesh,
        scratch_types=[
            pltpu.VMEM((CSUB,), jnp.int32),
            pltpu.VMEM((CSUB,), jnp.int32),
            pltpu.VMEM((CSUB,), jnp.int32),
            pltpu.VMEM((CSUB,), jnp.int32),
            pltpu.VMEM((CSUB, DIM), jnp.float32),
            pltpu.VMEM((CSUB, DIM), jnp.float32),
            pltpu.VMEM((CSUB, DIM), jnp.float32),
            pltpu.VMEM((CSUB, DIM), jnp.float32),
            pltpu.SemaphoreType.DMA,
            pltpu.SemaphoreType.DMA,
            pltpu.SemaphoreType.DMA,
        ],
    )
    def body(ys_hbm, pos_hbm, out_hbm,
             idx0a, idx1a, idx0b, idx1b, r0a, r1a, r0b, r1b,
             sga, sgb, sw):
        wid = lax.axis_index("s") * NCORES + lax.axis_index("c")
        base = wid * TPW
        bufs = [(idx0a, idx1a, r0a, r1a, sga), (idx0b, idx1b, r0b, r1b, sgb)]

        def start(j):
            i0, i1, r0, r1, sg = bufs[j % 2]
            b = base + j * CSUB
            pltpu.sync_copy(pos_hbm.at[pl.ds(b, CSUB)], i0)
            pltpu.sync_copy(pos_hbm.at[pl.ds(TOKENS + b, CSUB)], i1)
            return (pltpu.async_copy(ys_hbm.at[i0], r0, sg),
                    pltpu.async_copy(ys_hbm.at[i1], r1, sg))

        pend = start(0)
        wpend = None
        for j in range(nchunks):
            i0, i1, r0, r1, sg = bufs[j % 2]
            if wpend is not None:
                wpend.wait()
                wpend = None
            nxt = start(j + 1) if j + 1 < nchunks else None
            pend[0].wait()
            pend[1].wait()

            def inner(t, c2, r0=r0, r1=r1):
                for v in range(DIM // 16):
                    sl = pl.ds(v * 16, 16)
                    plsc.addupdate(r0.at[t, sl], r1[t, sl])
                return c2

            lax.fori_loop(0, CSUB, inner, 0)
            b = base + j * CSUB
            if j + 1 < nchunks:
                wpend = pltpu.async_copy(r0, out_hbm.at[pl.ds(b, CSUB)], sw)
            else:
                pltpu.sync_copy(r0, out_hbm.at[pl.ds(b, CSUB)])
            pend = nxt

    return body(ys, pos)


def _gmm_kernel(tb_ref, te_ref, tv_ref,
                xs_ref, w1_ref, b1_ref, w2_ref, b2_ref, ws_ref, ys_ref):
    t = pl.program_id(0)

    @pl.when(tv_ref[t] == 1)
    def _go():
        xb = xs_ref[...].astype(jnp.bfloat16)
        w1 = w1_ref[0].astype(jnp.bfloat16)
        w2 = w2_ref[0].astype(jnp.bfloat16)
        h = lax.dot_general(xb, w1, (((1,), (0,)), ((), ())),
                            preferred_element_type=jnp.float32) + b1_ref[0, 0][None, :]
        h = (h * 0.5 * (1.0 + lax.erf(h * 0.7071067811865476))).astype(jnp.bfloat16)
        y = lax.dot_general(h, w2, (((1,), (0,)), ((), ())),
                            preferred_element_type=jnp.float32) + b2_ref[0, 0][None, :]
        ys_ref[...] = y * ws_ref[:, 0:1]


def _grouped_mlp(tb, te, tv, xs, W1, b1, W2, b2, ws):
    grid_spec = pltpu.PrefetchScalarGridSpec(
        num_scalar_prefetch=3,
        grid=(NBP,),
        in_specs=[
            pl.BlockSpec((MB, DIM), lambda t, tb, te, tv: (tb[t], 0)),
            pl.BlockSpec((1, DIM, EXPERT_DIM),
                         lambda t, tb, te, tv: (te[t], 0, 0)),
            pl.BlockSpec((1, 1, EXPERT_DIM),
                         lambda t, tb, te, tv: (te[t], 0, 0)),
            pl.BlockSpec((1, EXPERT_DIM, DIM),
                         lambda t, tb, te, tv: (te[t], 0, 0)),
            pl.BlockSpec((1, 1, DIM),
                         lambda t, tb, te, tv: (te[t], 0, 0)),
            pl.BlockSpec((MB, 128), lambda t, tb, te, tv: (tb[t], 0)),
        ],
        out_specs=pl.BlockSpec((MB, DIM), lambda t, tb, te, tv: (tb[t], 0)),
    )
    return pl.pallas_call(
        _gmm_kernel,
        grid_spec=grid_spec,
        out_shape=jax.ShapeDtypeStruct((PADROWS, DIM), jnp.float32),
    )(tb, te, tv, xs, W1, b1[:, None, :], W2, b2[:, None, :], ws)


def kernel(x, routing_tensor, W1, b1, W2, b2):
    vals, eidx = lax.top_k(routing_tensor, TOP_K)
    e_flat = eidx.astype(jnp.int32).T.reshape(PAIRS)
    onehot = (e_flat[:, None] == jnp.arange(NUM_EXPERTS, dtype=jnp.int32)[None, :]
              ).astype(jnp.int32)
    counts = jnp.sum(onehot, axis=0)
    csum = jnp.cumsum(onehot, axis=0)
    rank = jnp.sum(onehot * (csum - 1), axis=1)

    bcnt = (counts + MB - 1) // MB
    cumb = jnp.cumsum(bcnt).astype(jnp.int32)
    boff = cumb - bcnt
    offp = (boff * MB).astype(jnp.int32)
    pos = (rank + jnp.sum(onehot * offp[None, :], axis=1)).astype(jnp.int32)

    ntp = cumb[-1]
    slot = jnp.arange(NBP, dtype=jnp.int32)
    teq = jnp.searchsorted(cumb, slot, side="right").astype(jnp.int32)
    tv = (slot < ntp).astype(jnp.int32)
    tb = jnp.where(slot < ntp, slot, ntp - 1)
    te = jnp.where(slot < ntp, teq, jnp.take(teq, ntp - 1))

    xs, ws_plane = _sc_dispatch(x, pos, vals[:, 0], vals[:, 1])
    ys = _grouped_mlp(tb, te, tv, xs, W1, b1, W2, b2, ws_plane)
    return _sc_combine(ys, pos)
